# Initial kernel scaffold; baseline (speedup 1.0000x reference)
import functools

import jax
import jax.numpy as jnp
from jax import lax
from jax.experimental import pallas as pl
from jax.experimental.pallas import tpu as pltpu
from jax.experimental.pallas import tpu_sc as plsc

BATCH = 16384
D = 128
LANES = 16
NUM_WORKERS = 32
ROWS_PER_W = BATCH // NUM_WORKERS
CHUNK = 128
NCHUNK = ROWS_PER_W // CHUNK
NGROUP = CHUNK // LANES

_mesh = plsc.VectorSubcoreMesh(core_axis_name="c", subcore_axis_name="s")


@functools.partial(
    pl.kernel,
    mesh=_mesh,
    out_type=jax.ShapeDtypeStruct((BATCH,), jnp.float32),
    scratch_types=[
        pltpu.VMEM((CHUNK,), jnp.int32),
        pltpu.VMEM((CHUNK,), jnp.int32),
        pltpu.VMEM((CHUNK, D), jnp.float32),
        pltpu.VMEM((CHUNK, D), jnp.float32),
        pltpu.VMEM((LANES * LANES,), jnp.float32),
        pltpu.VMEM((ROWS_PER_W,), jnp.float32),
        pltpu.SemaphoreType.DMA,
        pltpu.SemaphoreType.DMA,
    ],
)
def _mf_kernel(users, items, utab, itab, out,
               uidx, iidx, urows, irows, pbuf, outv, semu, semi):
    cid = lax.axis_index("c")
    sid = lax.axis_index("s")
    wid = sid * 2 + cid
    base = wid * ROWS_PER_W
    lanes16 = lax.iota(jnp.int32, LANES)

    def chunk_body(g, _):
        off = base + g * CHUNK
        pltpu.sync_copy(users.at[pl.ds(off, CHUNK)], uidx)
        pltpu.sync_copy(items.at[pl.ds(off, CHUNK)], iidx)
        cu = pltpu.async_copy(utab.at[uidx], urows, semu)
        ci = pltpu.async_copy(itab.at[iidx], irows, semi)
        cu.wait()
        ci.wait()

        def group_body(g2, _):
            row0 = g2 * LANES
            for r in range(LANES):
                row = row0 + r
                p = urows[row, pl.ds(0, LANES)] * irows[row, pl.ds(0, LANES)]
                for d in range(1, D // LANES):
                    p = p + (urows[row, pl.ds(d * LANES, LANES)]
                             * irows[row, pl.ds(d * LANES, LANES)])
                pbuf[pl.ds(r * LANES, LANES)] = p
            acc = plsc.load_gather(pbuf, [lanes16 * LANES])
            for l in range(1, LANES):
                acc = acc + plsc.load_gather(pbuf, [lanes16 * LANES + l])
            sig = 1.0 / (1.0 + jnp.exp(-acc))
            outv[pl.ds(g * CHUNK + row0, LANES)] = sig
            return 0

        lax.fori_loop(0, NGROUP, group_body, 0)
        return 0

    lax.fori_loop(0, NCHUNK, chunk_body, 0)
    pltpu.sync_copy(outv, out.at[pl.ds(base, ROWS_PER_W)])


def kernel(users, items, embedding_user, embedding_item):
    return _mf_kernel(users, items, embedding_user, embedding_item)


# speedup vs baseline: 1.1600x; 1.1600x over previous
import functools

import jax
import jax.numpy as jnp
from jax import lax
from jax.experimental import pallas as pl
from jax.experimental.pallas import tpu as pltpu
from jax.experimental.pallas import tpu_sc as plsc

BATCH = 16384
D = 128
LANES = 16
NUM_WORKERS = 32
ROWS_PER_W = BATCH // NUM_WORKERS
CHUNK = 128
NCHUNK = ROWS_PER_W // CHUNK
NGROUP = CHUNK // LANES

_mesh = plsc.VectorSubcoreMesh(core_axis_name="c", subcore_axis_name="s")


@functools.partial(
    pl.kernel,
    mesh=_mesh,
    out_type=jax.ShapeDtypeStruct((BATCH,), jnp.float32),
    compiler_params=pltpu.CompilerParams(needs_layout_passes=False),
    scratch_types=[
        pltpu.VMEM((CHUNK,), jnp.int32),
        pltpu.VMEM((CHUNK,), jnp.int32),
        pltpu.VMEM((CHUNK, D), jnp.float32),
        pltpu.VMEM((CHUNK, D), jnp.float32),
        pltpu.VMEM((LANES * LANES,), jnp.float32),
        pltpu.VMEM((ROWS_PER_W,), jnp.float32),
        pltpu.SemaphoreType.DMA,
        pltpu.SemaphoreType.DMA,
    ],
)
def _mf_kernel(users, items, utab, itab, out,
               uidx, iidx, urows, irows, pbuf, outv, semu, semi):
    cid = lax.axis_index("c")
    sid = lax.axis_index("s")
    wid = sid * 2 + cid
    base = wid * ROWS_PER_W
    lanes16 = lax.iota(jnp.int32, LANES)

    def chunk_body(g, _):
        off = base + g * CHUNK
        pltpu.sync_copy(users.at[pl.ds(off, CHUNK)], uidx)
        pltpu.sync_copy(items.at[pl.ds(off, CHUNK)], iidx)
        cu = pltpu.async_copy(utab.at[uidx], urows, semu)
        ci = pltpu.async_copy(itab.at[iidx], irows, semi)
        cu.wait()
        ci.wait()

        def group_body(g2, _):
            row0 = g2 * LANES
            for r in range(LANES):
                row = row0 + r
                p = urows[row, pl.ds(0, LANES)] * irows[row, pl.ds(0, LANES)]
                for d in range(1, D // LANES):
                    p = p + (urows[row, pl.ds(d * LANES, LANES)]
                             * irows[row, pl.ds(d * LANES, LANES)])
                pbuf[pl.ds(r * LANES, LANES)] = p
            acc = plsc.load_gather(pbuf, [lanes16 * LANES])
            for l in range(1, LANES):
                acc = acc + plsc.load_gather(pbuf, [lanes16 * LANES + l])
            sig = 1.0 / (1.0 + jnp.exp(-acc))
            outv[pl.ds(g * CHUNK + row0, LANES)] = sig
            return 0

        lax.fori_loop(0, NGROUP, group_body, 0)
        return 0

    lax.fori_loop(0, NCHUNK, chunk_body, 0)
    pltpu.sync_copy(outv, out.at[pl.ds(base, ROWS_PER_W)])


def kernel(users, items, embedding_user, embedding_item):
    return _mf_kernel(users, items, embedding_user, embedding_item)


# speedup vs baseline: 1.3156x; 1.1341x over previous
import functools

import jax
import jax.numpy as jnp
from jax import lax
from jax.experimental import pallas as pl
from jax.experimental.pallas import tpu as pltpu
from jax.experimental.pallas import tpu_sc as plsc

BATCH = 16384
D = 128
LANES = 16
NUM_WORKERS = 32
ROWS_PER_W = BATCH // NUM_WORKERS
CHUNK = 128
NCHUNK = ROWS_PER_W // CHUNK
NGROUP = CHUNK // LANES

_mesh = plsc.VectorSubcoreMesh(core_axis_name="c", subcore_axis_name="s")


@functools.partial(
    pl.kernel,
    mesh=_mesh,
    out_type=jax.ShapeDtypeStruct((BATCH,), jnp.float32),
    compiler_params=pltpu.CompilerParams(needs_layout_passes=False),
    scratch_types=[
        pltpu.VMEM((NCHUNK, CHUNK), jnp.int32),
        pltpu.VMEM((NCHUNK, CHUNK), jnp.int32),
        pltpu.VMEM((CHUNK, D), jnp.float32),
        pltpu.VMEM((CHUNK, D), jnp.float32),
        pltpu.VMEM((CHUNK, D), jnp.float32),
        pltpu.VMEM((CHUNK, D), jnp.float32),
        pltpu.VMEM((LANES * LANES,), jnp.float32),
        pltpu.VMEM((ROWS_PER_W,), jnp.float32),
        pltpu.SemaphoreType.DMA,
        pltpu.SemaphoreType.DMA,
        pltpu.SemaphoreType.DMA,
        pltpu.SemaphoreType.DMA,
    ],
)
def _mf_kernel(users, items, utab, itab, out,
               uidx, iidx, urows_a, irows_a, urows_b, irows_b, pbuf, outv,
               semu_a, semi_a, semu_b, semi_b):
    cid = lax.axis_index("c")
    sid = lax.axis_index("s")
    wid = sid * 2 + cid
    lanes16 = lax.iota(jnp.int32, LANES)

    pltpu.sync_copy(users.at[pl.ds(wid * NCHUNK, NCHUNK)], uidx)
    pltpu.sync_copy(items.at[pl.ds(wid * NCHUNK, NCHUNK)], iidx)

    bufs = [
        (urows_a, irows_a, semu_a, semi_a),
        (urows_b, irows_b, semu_b, semi_b),
    ]

    def start_gather(g):
        ur, ir, su, si = bufs[g % 2]
        cu = pltpu.async_copy(utab.at[uidx.at[g]], ur, su)
        ci = pltpu.async_copy(itab.at[iidx.at[g]], ir, si)
        return cu, ci

    def compute_chunk(g):
        ur, ir, _, _ = bufs[g % 2]

        def group_body(g2, _):
            row0 = g2 * LANES
            for r in range(LANES):
                row = row0 + r
                p = ur[row, pl.ds(0, LANES)] * ir[row, pl.ds(0, LANES)]
                for d in range(1, D // LANES):
                    p = p + (ur[row, pl.ds(d * LANES, LANES)]
                             * ir[row, pl.ds(d * LANES, LANES)])
                pbuf[pl.ds(r * LANES, LANES)] = p
            acc = plsc.load_gather(pbuf, [lanes16 * LANES])
            for l in range(1, LANES):
                acc = acc + plsc.load_gather(pbuf, [lanes16 * LANES + l])
            sig = 1.0 / (1.0 + jnp.exp(-acc))
            outv[pl.ds(g * CHUNK + row0, LANES)] = sig
            return 0

        lax.fori_loop(0, NGROUP, group_body, 0)

    copies = {0: start_gather(0)}
    for g in range(NCHUNK):
        if g + 1 < NCHUNK:
            copies[g + 1] = start_gather(g + 1)
        cu, ci = copies[g]
        cu.wait()
        ci.wait()
        compute_chunk(g)

    pltpu.sync_copy(outv, out.at[pl.ds(wid * ROWS_PER_W, ROWS_PER_W)])


def kernel(users, items, embedding_user, embedding_item):
    users2d = users.reshape(NUM_WORKERS * NCHUNK, CHUNK)
    items2d = items.reshape(NUM_WORKERS * NCHUNK, CHUNK)
    return _mf_kernel(users2d, items2d, embedding_user, embedding_item)
